# Initial kernel scaffold; baseline (speedup 1.0000x reference)
import jax
import jax.numpy as jnp
from jax.experimental import pallas as pl

_B, _N = 4096, 8192
_K = (_N - 1) // 2
_BR = 128
_IMIN = -(2 ** 31)


def _sbit(b: int) -> int:
    v = 1 << b
    return v - (1 << 32) if v >= (1 << 31) else v


def _median_block(x_ref, val_ref, idx_ref):
    x = x_ref[...]
    bits = jax.lax.bitcast_convert_type(x, jnp.int32)
    skey = bits ^ ((bits >> 31) & 0x7FFFFFFF)

    lo = jnp.zeros((_BR, 1), jnp.int32)
    kk = jnp.int32(_K)
    for b in range(31, -1, -1):
        trial_u = lo | jnp.int32(_sbit(b))
        trial_s = trial_u ^ jnp.int32(_IMIN)
        cnt = jnp.sum((skey < trial_s).astype(jnp.int32), axis=1,
                      keepdims=True)
        lo = jnp.where(cnt <= kk, trial_u, lo)

    vkey = lo ^ jnp.int32(_IMIN)
    cnt_less = jnp.sum((skey < vkey).astype(jnp.int32), axis=1,
                       keepdims=True)
    t = kk - cnt_less

    eq = (skey == vkey).astype(jnp.int32)
    iot = jax.lax.broadcasted_iota(jnp.int32, (_BR, _N), 1)
    ilo = jnp.zeros((_BR, 1), jnp.int32)
    for b in range(12, -1, -1):
        trial = ilo | jnp.int32(1 << b)
        cnt = jnp.sum(eq * (iot < trial).astype(jnp.int32), axis=1,
                      keepdims=True)
        ilo = jnp.where(cnt <= t, trial, ilo)

    vbits = vkey ^ ((vkey >> 31) & 0x7FFFFFFF)
    val_ref[...] = jax.lax.bitcast_convert_type(vbits, jnp.float32)
    idx_ref[...] = ilo


@jax.jit
def kernel(x):
    grid = (_B // _BR,)
    values, idx = pl.pallas_call(
        _median_block,
        grid=grid,
        in_specs=[pl.BlockSpec((_BR, _N), lambda i: (i, 0))],
        out_specs=[
            pl.BlockSpec((_BR, 1), lambda i: (i, 0)),
            pl.BlockSpec((_BR, 1), lambda i: (i, 0)),
        ],
        out_shape=[
            jax.ShapeDtypeStruct((_B, 1), jnp.float32),
            jax.ShapeDtypeStruct((_B, 1), jnp.int32),
        ],
    )(x)
    return values[:, 0], idx[:, 0]


# speedup vs baseline: 13.0752x; 13.0752x over previous
import jax
import jax.numpy as jnp
from jax.experimental import pallas as pl

_B, _N = 4096, 8192
_K = (_N - 1) // 2
_BR = 128
_IMIN = -(2 ** 31)


def _sbit(b: int) -> int:
    v = 1 << b
    return v - (1 << 32) if v >= (1 << 31) else v


def _median_block(x_ref, val_ref, idx_ref):
    x = x_ref[...]
    x = jnp.where(x == 0.0, jnp.float32(0.0), x)
    x = jnp.where(jnp.isnan(x), jnp.float32(jnp.nan), x)
    bits = jax.lax.bitcast_convert_type(x, jnp.int32)
    skey = bits ^ ((bits >> 31) & 0x7FFFFFFF)

    lo = jnp.zeros((_BR, 1), jnp.int32)
    kk = jnp.int32(_K)
    for b in range(31, -1, -1):
        trial_u = lo | jnp.int32(_sbit(b))
        trial_s = trial_u ^ jnp.int32(_IMIN)
        cnt = jnp.sum((skey < trial_s).astype(jnp.int32), axis=1,
                      keepdims=True)
        lo = jnp.where(cnt <= kk, trial_u, lo)

    vkey = lo ^ jnp.int32(_IMIN)
    cnt_less = jnp.sum((skey < vkey).astype(jnp.int32), axis=1,
                       keepdims=True)
    t = kk - cnt_less

    eq = (skey == vkey).astype(jnp.int32)
    iot = jax.lax.broadcasted_iota(jnp.int32, (_BR, _N), 1)
    ilo = jnp.zeros((_BR, 1), jnp.int32)
    for b in range(12, -1, -1):
        trial = ilo | jnp.int32(1 << b)
        cnt = jnp.sum(eq * (iot < trial).astype(jnp.int32), axis=1,
                      keepdims=True)
        ilo = jnp.where(cnt <= t, trial, ilo)

    vbits = vkey ^ ((vkey >> 31) & 0x7FFFFFFF)
    val_ref[...] = jax.lax.bitcast_convert_type(vbits, jnp.float32)
    idx_ref[...] = ilo


@jax.jit
def kernel(x):
    grid = (_B // _BR,)
    values, idx = pl.pallas_call(
        _median_block,
        grid=grid,
        in_specs=[pl.BlockSpec((_BR, _N), lambda i: (i, 0))],
        out_specs=[
            pl.BlockSpec((_BR, 1), lambda i: (i, 0)),
            pl.BlockSpec((_BR, 1), lambda i: (i, 0)),
        ],
        out_shape=[
            jax.ShapeDtypeStruct((_B, 1), jnp.float32),
            jax.ShapeDtypeStruct((_B, 1), jnp.int32),
        ],
    )(x)
    return values[:, 0], idx[:, 0]
